# Initial kernel scaffold; baseline (speedup 1.0000x reference)
import jax
import jax.numpy as jnp
from jax.experimental import pallas as pl


def kernel(prediction, target):
    raise NotImplementedError("write your pallas kernel here")


# speedup vs baseline: 10.2710x; 10.2710x over previous
import functools

import jax
import jax.numpy as jnp
from jax.experimental import pallas as pl

N_FEAT = 64
KMAX = 16
DV = 0.5
DD = 1.5
AL = 1.0
BE = 1.0
GA = 0.001


def _pass1_body(lab_ref, pred_ref, segsum_ref, counts_ref):
    i = pl.program_id(1)
    lab = lab_ref[0]
    pred = pred_ref[0]
    p = lab.shape[1]
    iota_k = jax.lax.broadcasted_iota(jnp.int32, (KMAX, p), 0)
    onehot = (iota_k == lab).astype(jnp.float32)
    part = jax.lax.dot_general(
        onehot, pred, (((1,), (1,)), ((), ())),
        preferred_element_type=jnp.float32)
    cnt = jnp.sum(onehot, axis=1, keepdims=True)

    @pl.when(i == 0)
    def _():
        segsum_ref[0] = part
        counts_ref[0] = cnt

    @pl.when(i > 0)
    def _():
        segsum_ref[0] += part
        counts_ref[0] += cnt


def _pass2_body(nblocks, lab_ref, pred_ref, segsum_ref, counts_ref,
                loss_ref, lvar_ref):
    b = pl.program_id(0)
    i = pl.program_id(1)
    lab = lab_ref[0]
    pred = pred_ref[0]
    segsum = segsum_ref[0]
    counts = counts_ref[0]
    p = lab.shape[1]

    ki = jnp.sum((counts > 0).astype(jnp.int32))
    kf = ki.astype(jnp.float32)
    iota_c = jax.lax.broadcasted_iota(jnp.int32, (KMAX, 1), 0)
    valid_c = iota_c < ki
    denom = jnp.where(valid_c, counts, 1.0)
    mu = jnp.where(valid_c, segsum / denom, 0.0)
    musq = jnp.sum(mu * mu, axis=1, keepdims=True)

    iota_k = jax.lax.broadcasted_iota(jnp.int32, (KMAX, p), 0)
    onehot = (iota_k == lab).astype(jnp.float32)
    lab_eff = jnp.minimum(lab, ki - 1)
    onehot_eff = (iota_k == lab_eff).astype(jnp.float32)

    dots = jax.lax.dot_general(
        mu, pred, (((1,), (0,)), ((), ())),
        preferred_element_type=jnp.float32)
    dot_p = jnp.sum(onehot_eff * dots, axis=0, keepdims=True)
    musq_p = jnp.sum(onehot_eff * musq, axis=0, keepdims=True)
    predsq = jnp.sum(pred * pred, axis=0, keepdims=True)
    dist = jnp.sqrt(jnp.maximum(musq_p - 2.0 * dot_p + predsq, 0.0))
    hinge = jnp.clip(dist - DV, 0.0, 10000.0)
    term = hinge * hinge
    lvar_part = jnp.sum(onehot * term, axis=1, keepdims=True)

    @pl.when(jnp.logical_and(b == 0, i == 0))
    def _():
        loss_ref[...] = jnp.zeros((1, 1), jnp.float32)

    @pl.when(i == 0)
    def _():
        lvar_ref[0] = lvar_part

    @pl.when(i > 0)
    def _():
        lvar_ref[0] += lvar_part

    @pl.when(i == nblocks - 1)
    def _():
        lvar_seg = lvar_ref[0]
        l_var = jnp.sum(jnp.where(valid_c, lvar_seg / (counts / kf), 0.0))

        gram = jax.lax.dot_general(
            mu, mu, (((1,), (1,)), ((), ())),
            preferred_element_type=jnp.float32)
        iota_r = jax.lax.broadcasted_iota(jnp.int32, (KMAX, KMAX), 0)
        iota_cc = jax.lax.broadcasted_iota(jnp.int32, (KMAX, KMAX), 1)
        eye = (iota_r == iota_cc).astype(jnp.float32)
        diag_col = jnp.sum(gram * eye, axis=1, keepdims=True)
        diag_row = jnp.sum(gram * eye, axis=0, keepdims=True)
        md = jnp.sqrt(jnp.maximum(diag_col + diag_row - 2.0 * gram, 0.0))
        aux = 2.0 * DD * (1.0 - eye)
        pair_valid = jnp.logical_and(iota_r < ki, iota_cc < ki)
        hd = jnp.clip(aux - md, 0.0, 10000.0)
        l_dist_mat = jnp.where(pair_valid, hd * hd, 0.0)
        l_dist = jnp.sum(l_dist_mat / (kf / (kf - 1.0)))

        l_reg = jnp.sum(jnp.sqrt(diag_col)) / kf

        loss_b = AL * l_var + BE * l_dist + GA * l_reg
        loss_ref[...] = loss_ref[...] + jnp.broadcast_to(loss_b, (1, 1))


@jax.jit
def kernel(prediction, target):
    B, H, W = target.shape
    hw = H * W
    P = 8192
    nblocks = hw // P
    pred = prediction.reshape(B, N_FEAT, hw)
    lab = target.reshape(B, 1, hw)

    segsum, counts = pl.pallas_call(
        _pass1_body,
        grid=(B, nblocks),
        in_specs=[
            pl.BlockSpec((1, 1, P), lambda b, i: (b, 0, i)),
            pl.BlockSpec((1, N_FEAT, P), lambda b, i: (b, 0, i)),
        ],
        out_specs=[
            pl.BlockSpec((1, KMAX, N_FEAT), lambda b, i: (b, 0, 0)),
            pl.BlockSpec((1, KMAX, 1), lambda b, i: (b, 0, 0)),
        ],
        out_shape=[
            jax.ShapeDtypeStruct((B, KMAX, N_FEAT), jnp.float32),
            jax.ShapeDtypeStruct((B, KMAX, 1), jnp.float32),
        ],
    )(lab, pred)

    loss, _ = pl.pallas_call(
        functools.partial(_pass2_body, nblocks),
        grid=(B, nblocks),
        in_specs=[
            pl.BlockSpec((1, 1, P), lambda b, i: (b, 0, i)),
            pl.BlockSpec((1, N_FEAT, P), lambda b, i: (b, 0, i)),
            pl.BlockSpec((1, KMAX, N_FEAT), lambda b, i: (b, 0, 0)),
            pl.BlockSpec((1, KMAX, 1), lambda b, i: (b, 0, 0)),
        ],
        out_specs=[
            pl.BlockSpec((1, 1), lambda b, i: (0, 0)),
            pl.BlockSpec((1, KMAX, 1), lambda b, i: (b, 0, 0)),
        ],
        out_shape=[
            jax.ShapeDtypeStruct((1, 1), jnp.float32),
            jax.ShapeDtypeStruct((B, KMAX, 1), jnp.float32),
        ],
    )(lab, pred, segsum, counts)

    return loss[0, 0]


# speedup vs baseline: 11.5860x; 1.1280x over previous
import functools

import jax
import jax.numpy as jnp
from jax.experimental import pallas as pl

N_FEAT = 64
KMAX = 16
DV = 0.5
DD = 1.5
AL = 1.0
BE = 1.0
GA = 0.001


def _pass1_body(lab_ref, pred_ref, segsum_ref, counts_ref):
    i = pl.program_id(1)
    lab = lab_ref[0]
    pred = pred_ref[0]
    p = lab.shape[1]
    iota_k = jax.lax.broadcasted_iota(jnp.int32, (KMAX, p), 0)
    onehot = (iota_k == lab).astype(jnp.float32)
    part = jax.lax.dot_general(
        onehot, pred, (((1,), (1,)), ((), ())),
        preferred_element_type=jnp.float32)
    cnt = jnp.sum(onehot, axis=1, keepdims=True)

    @pl.when(i == 0)
    def _():
        segsum_ref[0] = part
        counts_ref[0] = cnt

    @pl.when(i > 0)
    def _():
        segsum_ref[0] += part
        counts_ref[0] += cnt


def _pass2_body(nblocks, lab_ref, pred_ref, segsum_ref, counts_ref,
                loss_ref, lvar_ref):
    b = pl.program_id(0)
    i = pl.program_id(1)
    lab = lab_ref[0]
    pred = pred_ref[0]
    segsum = segsum_ref[0]
    counts = counts_ref[0]
    p = lab.shape[1]

    ki = jnp.sum((counts > 0).astype(jnp.int32))
    kf = ki.astype(jnp.float32)
    iota_c = jax.lax.broadcasted_iota(jnp.int32, (KMAX, 1), 0)
    valid_c = iota_c < ki
    denom = jnp.where(valid_c, counts, 1.0)
    mu = jnp.where(valid_c, segsum / denom, 0.0)
    musq = jnp.sum(mu * mu, axis=1, keepdims=True)

    iota_k = jax.lax.broadcasted_iota(jnp.int32, (KMAX, p), 0)
    onehot = (iota_k == lab).astype(jnp.float32)
    lab_eff = jnp.minimum(lab, ki - 1)
    onehot_eff = (iota_k == lab_eff).astype(jnp.float32)

    dots = jax.lax.dot_general(
        mu, pred, (((1,), (0,)), ((), ())),
        preferred_element_type=jnp.float32)
    dot_p = jnp.sum(onehot_eff * dots, axis=0, keepdims=True)
    musq_p = jnp.sum(onehot_eff * musq, axis=0, keepdims=True)
    predsq = jnp.sum(pred * pred, axis=0, keepdims=True)
    dist = jnp.sqrt(jnp.maximum(musq_p - 2.0 * dot_p + predsq, 0.0))
    hinge = jnp.clip(dist - DV, 0.0, 10000.0)
    term = hinge * hinge
    lvar_part = jnp.sum(onehot * term, axis=1, keepdims=True)

    @pl.when(jnp.logical_and(b == 0, i == 0))
    def _():
        loss_ref[...] = jnp.zeros((1, 1), jnp.float32)

    @pl.when(i == 0)
    def _():
        lvar_ref[0] = lvar_part

    @pl.when(i > 0)
    def _():
        lvar_ref[0] += lvar_part

    @pl.when(i == nblocks - 1)
    def _():
        lvar_seg = lvar_ref[0]
        l_var = jnp.sum(jnp.where(valid_c, lvar_seg / (counts / kf), 0.0))

        gram = jax.lax.dot_general(
            mu, mu, (((1,), (1,)), ((), ())),
            preferred_element_type=jnp.float32)
        iota_r = jax.lax.broadcasted_iota(jnp.int32, (KMAX, KMAX), 0)
        iota_cc = jax.lax.broadcasted_iota(jnp.int32, (KMAX, KMAX), 1)
        eye = (iota_r == iota_cc).astype(jnp.float32)
        diag_col = jnp.sum(gram * eye, axis=1, keepdims=True)
        diag_row = jnp.sum(gram * eye, axis=0, keepdims=True)
        md = jnp.sqrt(jnp.maximum(diag_col + diag_row - 2.0 * gram, 0.0))
        aux = 2.0 * DD * (1.0 - eye)
        pair_valid = jnp.logical_and(iota_r < ki, iota_cc < ki)
        hd = jnp.clip(aux - md, 0.0, 10000.0)
        l_dist_mat = jnp.where(pair_valid, hd * hd, 0.0)
        l_dist = jnp.sum(l_dist_mat / (kf / (kf - 1.0)))

        l_reg = jnp.sum(jnp.sqrt(diag_col)) / kf

        loss_b = AL * l_var + BE * l_dist + GA * l_reg
        loss_ref[...] = loss_ref[...] + jnp.broadcast_to(loss_b, (1, 1))


@jax.jit
def kernel(prediction, target):
    B, H, W = target.shape
    hw = H * W
    P = 16384
    nblocks = hw // P
    pred = prediction.reshape(B, N_FEAT, hw)
    lab = target.reshape(B, 1, hw)

    segsum, counts = pl.pallas_call(
        _pass1_body,
        grid=(B, nblocks),
        in_specs=[
            pl.BlockSpec((1, 1, P), lambda b, i: (b, 0, i)),
            pl.BlockSpec((1, N_FEAT, P), lambda b, i: (b, 0, i)),
        ],
        out_specs=[
            pl.BlockSpec((1, KMAX, N_FEAT), lambda b, i: (b, 0, 0)),
            pl.BlockSpec((1, KMAX, 1), lambda b, i: (b, 0, 0)),
        ],
        out_shape=[
            jax.ShapeDtypeStruct((B, KMAX, N_FEAT), jnp.float32),
            jax.ShapeDtypeStruct((B, KMAX, 1), jnp.float32),
        ],
    )(lab, pred)

    loss, _ = pl.pallas_call(
        functools.partial(_pass2_body, nblocks),
        grid=(B, nblocks),
        in_specs=[
            pl.BlockSpec((1, 1, P), lambda b, i: (b, 0, i)),
            pl.BlockSpec((1, N_FEAT, P), lambda b, i: (b, 0, i)),
            pl.BlockSpec((1, KMAX, N_FEAT), lambda b, i: (b, 0, 0)),
            pl.BlockSpec((1, KMAX, 1), lambda b, i: (b, 0, 0)),
        ],
        out_specs=[
            pl.BlockSpec((1, 1), lambda b, i: (0, 0)),
            pl.BlockSpec((1, KMAX, 1), lambda b, i: (b, 0, 0)),
        ],
        out_shape=[
            jax.ShapeDtypeStruct((1, 1), jnp.float32),
            jax.ShapeDtypeStruct((B, KMAX, 1), jnp.float32),
        ],
    )(lab, pred, segsum, counts)

    return loss[0, 0]


# speedup vs baseline: 13.2824x; 1.1464x over previous
import functools

import jax
import jax.numpy as jnp
from jax.experimental import pallas as pl
from jax.experimental.pallas import tpu as pltpu

N_FEAT = 64
KMAX = 16
DV = 0.5
DD = 1.5
AL = 1.0
BE = 1.0
GA = 0.001


def _body(nb, nimg, lab_ref, pred_ref, loss_ref,
          predbf_st, predsq_st, lab_st, segsum_st, counts_st, lvar_st):
    b = pl.program_id(0)
    i = pl.program_id(1)
    slot = jax.lax.rem(b, 2)

    @pl.when(jnp.logical_and(b == 0, i == 0))
    def _():
        loss_ref[...] = jnp.zeros((1, 1), jnp.float32)

    @pl.when(b < nimg)
    def _phase0():
        lab = lab_ref[0]
        pred = pred_ref[0]
        p = lab.shape[1]
        iota_k = jax.lax.broadcasted_iota(jnp.int32, (KMAX, p), 0)
        onehot = (iota_k == lab).astype(jnp.float32)
        part = jax.lax.dot_general(
            onehot, pred, (((1,), (1,)), ((), ())),
            preferred_element_type=jnp.float32)
        cnt = jnp.sum(onehot, axis=1, keepdims=True)
        predbf_st[slot, i] = pred.astype(jnp.bfloat16)
        predsq_st[slot, i] = jnp.sum(pred * pred, axis=0, keepdims=True)
        lab_st[slot, i] = lab

        @pl.when(i == 0)
        def _():
            segsum_st[slot] = part
            counts_st[slot] = cnt

        @pl.when(i > 0)
        def _():
            segsum_st[slot] = segsum_st[slot] + part
            counts_st[slot] = counts_st[slot] + cnt

    @pl.when(b > 0)
    def _phase1():
        s2 = 1 - slot
        lab = lab_st[s2, i]
        predbf = predbf_st[s2, i]
        predsq = predsq_st[s2, i]
        segsum = segsum_st[s2]
        counts = counts_st[s2]
        p = lab.shape[1]

        ki = jnp.sum((counts > 0).astype(jnp.int32))
        kf = ki.astype(jnp.float32)
        iota_c = jax.lax.broadcasted_iota(jnp.int32, (KMAX, 1), 0)
        valid_c = iota_c < ki
        denom = jnp.where(valid_c, counts, 1.0)
        mu = jnp.where(valid_c, segsum / denom, 0.0)
        musq = jnp.sum(mu * mu, axis=1, keepdims=True)

        iota_k = jax.lax.broadcasted_iota(jnp.int32, (KMAX, p), 0)
        onehot = (iota_k == lab).astype(jnp.float32)
        lab_eff = jnp.minimum(lab, ki - 1)
        onehot_eff = (iota_k == lab_eff).astype(jnp.float32)

        dots = jax.lax.dot_general(
            mu.astype(jnp.bfloat16), predbf, (((1,), (0,)), ((), ())),
            preferred_element_type=jnp.float32)
        dot_p = jnp.sum(onehot_eff * dots, axis=0, keepdims=True)
        musq_p = jnp.sum(onehot_eff * musq, axis=0, keepdims=True)
        dist = jnp.sqrt(jnp.maximum(musq_p - 2.0 * dot_p + predsq, 0.0))
        hinge = jnp.clip(dist - DV, 0.0, 10000.0)
        term = hinge * hinge
        lvar_part = jnp.sum(onehot * term, axis=1, keepdims=True)

        @pl.when(i == 0)
        def _():
            lvar_st[s2] = lvar_part

        @pl.when(i > 0)
        def _():
            lvar_st[s2] = lvar_st[s2] + lvar_part

        @pl.when(i == nb - 1)
        def _():
            lvar_seg = lvar_st[s2]
            l_var = jnp.sum(jnp.where(valid_c, lvar_seg / (counts / kf), 0.0))

            gram = jax.lax.dot_general(
                mu, mu, (((1,), (1,)), ((), ())),
                preferred_element_type=jnp.float32)
            iota_r = jax.lax.broadcasted_iota(jnp.int32, (KMAX, KMAX), 0)
            iota_cc = jax.lax.broadcasted_iota(jnp.int32, (KMAX, KMAX), 1)
            eye = (iota_r == iota_cc).astype(jnp.float32)
            diag_col = jnp.sum(gram * eye, axis=1, keepdims=True)
            diag_row = jnp.sum(gram * eye, axis=0, keepdims=True)
            md = jnp.sqrt(jnp.maximum(diag_col + diag_row - 2.0 * gram, 0.0))
            aux = 2.0 * DD * (1.0 - eye)
            pair_valid = jnp.logical_and(iota_r < ki, iota_cc < ki)
            hd = jnp.clip(aux - md, 0.0, 10000.0)
            l_dist = jnp.sum(jnp.where(pair_valid, hd * hd, 0.0)
                             / (kf / (kf - 1.0)))
            l_reg = jnp.sum(jnp.sqrt(diag_col)) / kf

            loss_b = AL * l_var + BE * l_dist + GA * l_reg
            loss_ref[...] = loss_ref[...] + jnp.broadcast_to(loss_b, (1, 1))


@jax.jit
def kernel(prediction, target):
    B, H, W = target.shape
    hw = H * W
    P = 16384
    nb = hw // P
    pred = prediction.reshape(B, N_FEAT, hw)
    lab = target.reshape(B, 1, hw)

    def idx(b, i):
        img = jnp.minimum(b, B - 1)
        blk = jnp.where(b < B, i, nb - 1)
        return (img, 0, blk)

    loss = pl.pallas_call(
        functools.partial(_body, nb, B),
        grid=(B + 1, nb),
        in_specs=[
            pl.BlockSpec((1, 1, P), idx),
            pl.BlockSpec((1, N_FEAT, P), idx),
        ],
        out_specs=pl.BlockSpec((1, 1), lambda b, i: (0, 0)),
        out_shape=jax.ShapeDtypeStruct((1, 1), jnp.float32),
        scratch_shapes=[
            pltpu.VMEM((2, nb, N_FEAT, P), jnp.bfloat16),
            pltpu.VMEM((2, nb, 1, P), jnp.float32),
            pltpu.VMEM((2, nb, 1, P), jnp.int32),
            pltpu.VMEM((2, KMAX, N_FEAT), jnp.float32),
            pltpu.VMEM((2, KMAX, 1), jnp.float32),
            pltpu.VMEM((2, KMAX, 1), jnp.float32),
        ],
        compiler_params=pltpu.CompilerParams(
            vmem_limit_bytes=100 * 1024 * 1024,
        ),
    )(lab, pred)

    return loss[0, 0]


# speedup vs baseline: 13.3545x; 1.0054x over previous
import functools

import jax
import jax.numpy as jnp
from jax.experimental import pallas as pl
from jax.experimental.pallas import tpu as pltpu

N_FEAT = 64
NAUG = 80
KMAX = 16
DV = 0.5
DD = 1.5
AL = 1.0
BE = 1.0
GA = 0.001


def _body(nb, nimg, lab0_ref, lab1_ref, pred_ref, loss_ref,
          aug_st, predsq_st, segsum_st, lvar_st):
    b = pl.program_id(0)
    i = pl.program_id(1)
    slot = jax.lax.rem(b, 2)

    @pl.when(jnp.logical_and(b == 0, i == 0))
    def _():
        loss_ref[...] = jnp.zeros((1, 1), jnp.float32)

    @pl.when(b < nimg)
    def _phase0():
        lab = lab0_ref[0]
        pred = pred_ref[0]
        p = lab.shape[1]
        aug = jnp.concatenate(
            [pred.astype(jnp.bfloat16),
             jnp.ones((1, p), jnp.bfloat16),
             jnp.zeros((NAUG - N_FEAT - 1, p), jnp.bfloat16)], axis=0)
        aug_st[slot, i] = aug
        predsq_st[slot, i] = jnp.sum(pred * pred, axis=0, keepdims=True)

        iota_k = jax.lax.broadcasted_iota(jnp.int32, (KMAX, p), 0)
        onehot = (iota_k == lab).astype(jnp.float32).astype(jnp.bfloat16)
        part = jax.lax.dot_general(
            onehot, aug, (((1,), (1,)), ((), ())),
            preferred_element_type=jnp.float32)

        @pl.when(i == 0)
        def _():
            segsum_st[slot] = part

        @pl.when(i > 0)
        def _():
            segsum_st[slot] = segsum_st[slot] + part

    @pl.when(b > 0)
    def _phase1():
        s2 = 1 - slot
        lab = lab1_ref[0]
        aug = aug_st[s2, i]
        predsq = predsq_st[s2, i]
        segsum = segsum_st[s2, :, 0:N_FEAT]
        counts = segsum_st[s2, :, N_FEAT:N_FEAT + 1]
        p = lab.shape[1]

        ki = jnp.sum((counts > 0).astype(jnp.int32))
        kf = ki.astype(jnp.float32)
        iota_c = jax.lax.broadcasted_iota(jnp.int32, (KMAX, 1), 0)
        valid_c = iota_c < ki
        denom = jnp.where(valid_c, counts, 1.0)
        mu = jnp.where(valid_c, segsum / denom, 0.0)
        musq = jnp.sum(mu * mu, axis=1, keepdims=True)

        amat = jnp.concatenate(
            [(-2.0 * mu).astype(jnp.bfloat16),
             musq.astype(jnp.bfloat16),
             jnp.zeros((KMAX, NAUG - N_FEAT - 1), jnp.bfloat16)], axis=1)
        xmat = jax.lax.dot_general(
            amat, aug, (((1,), (0,)), ((), ())),
            preferred_element_type=jnp.float32)

        iota_k = jax.lax.broadcasted_iota(jnp.int32, (KMAX, p), 0)
        is_own = iota_k == lab
        sel = jnp.sum(jnp.where(is_own, xmat, 0.0), axis=0, keepdims=True)
        dist = jnp.sqrt(jnp.maximum(sel + predsq, 0.0))
        hinge = jnp.clip(dist - DV, 0.0, 10000.0)
        term = hinge * hinge

        onehot = is_own.astype(jnp.float32).astype(jnp.bfloat16)
        lvar_part = jax.lax.dot_general(
            term, onehot, (((1,), (1,)), ((), ())),
            preferred_element_type=jnp.float32)

        @pl.when(i == 0)
        def _():
            lvar_st[s2] = lvar_part

        @pl.when(i > 0)
        def _():
            lvar_st[s2] = lvar_st[s2] + lvar_part

        @pl.when(i == nb - 1)
        def _():
            lvar_row = lvar_st[s2]
            w_col = jnp.where(valid_c, kf / counts, 0.0)
            l_var = jax.lax.dot_general(
                lvar_row, w_col, (((1,), (0,)), ((), ())),
                preferred_element_type=jnp.float32)[0, 0]

            gram = jax.lax.dot_general(
                mu, mu, (((1,), (1,)), ((), ())),
                preferred_element_type=jnp.float32)
            iota_r = jax.lax.broadcasted_iota(jnp.int32, (KMAX, KMAX), 0)
            iota_cc = jax.lax.broadcasted_iota(jnp.int32, (KMAX, KMAX), 1)
            eye = (iota_r == iota_cc).astype(jnp.float32)
            diag_col = jnp.sum(gram * eye, axis=1, keepdims=True)
            diag_row = jnp.sum(gram * eye, axis=0, keepdims=True)
            md = jnp.sqrt(jnp.maximum(diag_col + diag_row - 2.0 * gram, 0.0))
            aux = 2.0 * DD * (1.0 - eye)
            pair_valid = jnp.logical_and(iota_r < ki, iota_cc < ki)
            hd = jnp.clip(aux - md, 0.0, 10000.0)
            l_dist = jnp.sum(jnp.where(pair_valid, hd * hd, 0.0)
                             / (kf / (kf - 1.0)))
            l_reg = jnp.sum(jnp.sqrt(diag_col)) / kf

            loss_b = AL * l_var + BE * l_dist + GA * l_reg
            loss_ref[...] = loss_ref[...] + jnp.broadcast_to(loss_b, (1, 1))


@jax.jit
def kernel(prediction, target):
    B, H, W = target.shape
    hw = H * W
    P = 16384
    nb = hw // P
    pred = prediction.reshape(B, N_FEAT, hw)
    lab = target.reshape(B, 1, hw)

    def idx0(b, i):
        img = jnp.minimum(b, B - 1)
        blk = jnp.where(b < B, i, nb - 1)
        return (img, 0, blk)

    def idx1(b, i):
        img = jnp.maximum(b, 1) - 1
        blk = jnp.where(b > 0, i, 0)
        return (img, 0, blk)

    loss = pl.pallas_call(
        functools.partial(_body, nb, B),
        grid=(B + 1, nb),
        in_specs=[
            pl.BlockSpec((1, 1, P), idx0),
            pl.BlockSpec((1, 1, P), idx1),
            pl.BlockSpec((1, N_FEAT, P), idx0),
        ],
        out_specs=pl.BlockSpec((1, 1), lambda b, i: (0, 0)),
        out_shape=jax.ShapeDtypeStruct((1, 1), jnp.float32),
        scratch_shapes=[
            pltpu.VMEM((2, nb, NAUG, P), jnp.bfloat16),
            pltpu.VMEM((2, nb, 1, P), jnp.float32),
            pltpu.VMEM((2, KMAX, NAUG), jnp.float32),
            pltpu.VMEM((2, 1, KMAX), jnp.float32),
        ],
        compiler_params=pltpu.CompilerParams(
            vmem_limit_bytes=100 * 1024 * 1024,
        ),
    )(lab, lab, pred)

    return loss[0, 0]
